# baseline (speedup 1.0000x reference)
import jax
import jax.numpy as jnp
from jax import lax
from jax.experimental import pallas as pl
from jax.experimental.pallas import tpu as pltpu
from jax.experimental.pallas import tpu_sc as plsc

_B, _C, _N = 8, 3, 2048
_DK, _K = 100, 20
_R = 128
_NW = 32
_ROWS_W = _B * _N // _NW
_BATCH = 4
_NBAT = _ROWS_W // _BATCH
_OW = 128 + 480
_NW2 = _N + 16
_POS = float("inf")


def _split3(a):
    hi = a.astype(jnp.bfloat16)
    r1 = a - hi.astype(jnp.float32)
    mid = r1.astype(jnp.bfloat16)
    lo = (r1 - mid.astype(jnp.float32)).astype(jnp.bfloat16)
    return hi, mid, lo


def _pd_body(x_ref, pd_ref):
    t = pl.program_id(1)
    xb = x_ref[0]
    xx_cols = xb[0:1] * xb[0:1] + xb[1:2] * xb[1:2] + xb[2:3] * xb[2:3]

    lane_n = lax.broadcasted_iota(jnp.int32, (_R, _N), 1)
    row_r = lax.broadcasted_iota(jnp.int32, (_R, 1), 0)

    xhi, xmid, xlo = _split3(xb)
    qhi, qmid, qlo = _split3(xx_cols)
    x12 = jnp.concatenate([xhi, xmid, xlo, qhi, qmid, qlo], axis=0)

    oh_self = (lane_n == t * _R + row_r).astype(jnp.bfloat16)
    g12 = lax.dot_general(oh_self, x12, (((1,), (1,)), ((), ())),
                          preferred_element_type=jnp.float32)
    xtT = g12[:, 0:3] + g12[:, 3:6] + g12[:, 6:9]
    xx_rows = g12[:, 9:10] + g12[:, 10:11] + g12[:, 11:12]

    inner = -2.0 * lax.dot_general(xtT.astype(jnp.bfloat16), xb.astype(jnp.bfloat16),
                                   (((1,), (0,)), ((), ())),
                                   preferred_element_type=jnp.float32)
    q = (xx_cols + inner) + xx_rows

    q3 = q.reshape(_R, 16, 128)
    for _ in range(8):
        m = jnp.min(q3, axis=2, keepdims=True)
        q3 = jnp.where(q3 == m, _POS, q3)
    tau = jnp.max(m.reshape(_R, 16), axis=1, keepdims=True)
    pd_ref[0, :, 0:_N] = q
    pd_ref[0, :, _N:_NW2] = jnp.broadcast_to(tau, (_R, 16))


def _rev(v):
    return lax.rev(v, (0,))


def _sort16(v, i):
    return lax.sort((v, i), dimension=0, num_keys=1, is_stable=False)


def _cmpsel(av, ai, bv, bi):
    bw = (bv < av) | ((bv == av) & (bi < ai))
    hv = jnp.where(bw, bv, av)
    hi_ = jnp.where(bw, bi, ai)
    lv = jnp.where(bw, av, bv)
    li = jnp.where(bw, ai, bi)
    return hv, hi_, lv, li


def _bmerge(vs, js):
    m = len(vs)
    if m == 1:
        v, j = _sort16(vs[0], js[0])
        return [v], [j]
    h = m // 2
    hv, hj, lv, lj = [], [], [], []
    for k in range(h):
        a, b, c, d = _cmpsel(vs[k], js[k], vs[k + h], js[k + h])
        hv.append(a); hj.append(b); lv.append(c); lj.append(d)
    rv1, rj1 = _bmerge(hv, hj)
    rv2, rj2 = _bmerge(lv, lj)
    return rv1 + rv2, rj1 + rj2


def _merge(av, aj, bv, bj):
    m = len(av)
    brv = [_rev(bv[m - 1 - k]) for k in range(m)]
    brj = [_rev(bj[m - 1 - k]) for k in range(m)]
    hv, hj, lv, lj = [], [], [], []
    for k in range(m):
        a, b, c, d = _cmpsel(av[k], aj[k], brv[k], brj[k])
        hv.append(a); hj.append(b); lv.append(c); lj.append(d)
    rv1, rj1 = _bmerge(hv, hj)
    rv2, rj2 = _bmerge(lv, lj)
    return rv1 + rv2, rj1 + rj2


def _sortN(vs, js):
    n = len(vs)
    if n == 1:
        v, j = _sort16(vs[0], js[0])
        return [v], [j]
    h = n // 2
    av, aj = _sortN(vs[:h], js[:h])
    bv, bj = _sortN(vs[h:], js[h:])
    return _merge(av, aj, bv, bj)


def _prune_merge(sv, sj, cv, cj):
    hv, hj = [], []
    for k in range(8):
        a, b, _, _ = _cmpsel(sv[k], sj[k], _rev(cv[7 - k]), _rev(cj[7 - k]))
        hv.append(a); hj.append(b)
    return _bmerge(hv, hj)


def _sc_body(pd_hbm, x_hbm, out_hbm,
             rowbuf, candv, candi, st, si_st, xbuf, sem_in, sem_out):
    wid = lax.axis_index("s") * 2 + lax.axis_index("c")
    row0 = wid * _ROWS_W
    b = row0 // _N
    pltpu.sync_copy(x_hbm.at[b], xbuf)

    iota16 = lax.iota(jnp.int32, 16)

    def cp_in(i, par):
        return pltpu.make_async_copy(
            pd_hbm.at[pl.ds((row0 + _BATCH * i) * _NW2, _BATCH * _NW2)],
            rowbuf.at[pl.ds(par * _BATCH * _NW2, _BATCH * _NW2)], sem_in)

    def cp_out(i, par):
        return pltpu.make_async_copy(
            st.at[pl.ds(par * _BATCH * _OW, _BATCH * _OW)],
            out_hbm.at[pl.ds((row0 + _BATCH * i) * _OW, _BATCH * _OW)], sem_out)

    def prefill():
        for k in range(8):
            candv[pl.ds(16 * k, 16)] = jnp.full((16,), _POS, jnp.float32)
            candi[pl.ds(16 * k, 16)] = jnp.full((16,), _N - 1, jnp.int32)

    def flat(sv, sj):
        return tuple(sv) + tuple(sj)

    def unflat(t):
        return list(t[:8]), list(t[8:16])

    def process_row(par, q):
        rbase = par * _BATCH * _NW2 + q * _NW2
        sv = [jnp.full((16,), _POS, jnp.float32) for _ in range(8)]
        sj = [jnp.full((16,), _N - 1, jnp.int32) for _ in range(8)]
        prefill()
        tauvec = rowbuf[pl.ds(rbase + _N, 16)]
        tau = tauvec[0]

        def do_merge(args):
            t8, _tau, _cnt = args
            svx, sjx = unflat(t8)
            cv = [candv[pl.ds(16 * k, 16)] for k in range(8)]
            ci = [candi[pl.ds(16 * k, 16)] for k in range(8)]
            cv, ci = _sortN(cv, ci)
            svx, sjx = _prune_merge(svx, sjx, cv, ci)
            prefill()
            return flat(svx, sjx), jnp.max(svx[7]), jnp.int32(0)

        def blk(j, carry):
            t8, tau_c, cnt = carry
            v = rowbuf[pl.ds(rbase + 16 * j, 16)]
            iv = iota16 + 16 * j
            mask = v <= tau_c
            pos = plsc.cumsum(mask.astype(jnp.int32))
            tgt = cnt + pos - 1
            plsc.store_scatter(candv, [tgt], v, mask=mask)
            plsc.store_scatter(candi, [tgt], iv, mask=mask)
            cnt = cnt + pos[15]
            return lax.cond(cnt >= 112, do_merge, lambda a: a, (t8, tau_c, cnt))

        carry = lax.fori_loop(0, _N // 16, blk,
                              (flat(sv, sj), tau, jnp.int32(0)), unroll=4)
        t8, _, _ = lax.cond(carry[2] > 0, do_merge, lambda a: a, carry)
        sv, sj = unflat(t8)

        obase = par * _BATCH * _OW + q * _OW
        for k in range(8):
            st[pl.ds(obase + 16 * k, 16)] = sv[k]
            si_st[pl.ds(16 * k, 16)] = sj[k]

        for v in range(1, 6):
            for grp in range(2):
                posv = jnp.minimum((iota16 + 16 * grp) * v, 127)
                nidx = plsc.load_gather(si_st, [posv])
                for c in range(_C):
                    cvec = jnp.full((16,), c, jnp.int32)
                    xs = plsc.load_gather(xbuf, [cvec, nidx])
                    st[pl.ds(obase + 128 + (v - 1) * 96 + c * 32 + grp * 16, 16)] = xs

    cp_in(0, 0).start()

    def batch(i, _):
        par = lax.rem(i, 2)

        @pl.when(i + 1 < _NBAT)
        def _():
            cp_in(i + 1, 1 - par).start()

        cp_in(i, par).wait()

        @pl.when(i >= 2)
        def _():
            cp_out(i - 2, par).wait()

        def inner(q, __):
            process_row(par, q)
            return 0

        lax.fori_loop(0, _BATCH, inner, 0)
        cp_out(i, par).start()
        return 0

    lax.fori_loop(0, _NBAT, batch, 0)
    cp_out(_NBAT - 2, lax.rem(jnp.int32(_NBAT - 2), 2)).wait()
    cp_out(_NBAT - 1, lax.rem(jnp.int32(_NBAT - 1), 2)).wait()


def _ec_body(x_ref, sc_ref, w1_ref, w11_ref, wc_ref, g_ref, b_ref, out_ref):
    t = pl.program_id(1)
    xb = x_ref[0]
    lane_n = lax.broadcasted_iota(jnp.int32, (_R, _N), 1)
    row_r = lax.broadcasted_iota(jnp.int32, (_R, 1), 0)
    xhi, xmid, xlo = _split3(xb)
    x9 = jnp.concatenate([xhi, xmid, xlo], axis=0)
    oh_self = (lane_n == t * _R + row_r).astype(jnp.bfloat16)
    g9 = lax.dot_general(oh_self, x9, (((1,), (1,)), ((), ())),
                         preferred_element_type=jnp.float32)
    xtT = g9[:, 0:3] + g9[:, 3:6] + g9[:, 6:9]

    metric = sc_ref[0][:, 0:128]
    w1pad = jnp.concatenate(
        [w1_ref[...], jnp.zeros((64, 128 - _DK), jnp.float32)], axis=1)
    m1 = lax.dot_general(metric.astype(jnp.bfloat16), w1pad.astype(jnp.bfloat16),
                         (((1,), (1,)), ((), ())),
                         preferred_element_type=jnp.float32)
    w11pad = jnp.concatenate(
        [w11_ref[...], jnp.zeros((7, 64), jnp.float32)], axis=0)
    m2 = lax.dot_general(m1.astype(jnp.bfloat16), w11pad.astype(jnp.bfloat16),
                         (((1,), (1,)), ((), ())),
                         preferred_element_type=jnp.float32)[:, 0:1]
    ms = 5.0 * jax.nn.sigmoid(-m2) + 0.5
    value = (jnp.where((ms >= 0.5) & (ms < 1.5), 1.0, 0.0)
             + jnp.where((ms >= 1.5) & (ms < 2.5), 2.0, 0.0)
             + jnp.where((ms >= 2.5) & (ms < 3.5), 3.0, 0.0)
             + jnp.where((ms >= 3.5) & (ms < 4.5), 4.0, 0.0)
             + jnp.where((ms >= 4.5) & (ms <= 5.5), 5.0, 0.0))

    nb = sc_ref[0][:, 128:_OW]
    sel = jnp.zeros((_R, 96), jnp.float32)
    for v in range(1, 6):
        sel = jnp.where(value == jnp.float32(v), nb[:, 96 * (v - 1):96 * v], sel)

    wc_bf = wc_ref[...].astype(jnp.bfloat16)
    gamma = g_ref[...]
    beta = b_ref[...]
    acc = jnp.full((_R, 64), float("-inf"), jnp.float32)
    for k in range(_K):
        n0 = sel[:, k:k + 1]
        n1 = sel[:, 32 + k:33 + k]
        n2 = sel[:, 64 + k:65 + k]
        feat = jnp.concatenate(
            [n0 - xtT[:, 0:1], n1 - xtT[:, 1:2], n2 - xtT[:, 2:3], xtT], axis=1)
        h = lax.dot_general(feat.astype(jnp.bfloat16), wc_bf,
                            (((1,), (1,)), ((), ())),
                            preferred_element_type=jnp.float32)
        h = h * gamma + beta
        h = jnp.where(h >= 0, h, 0.2 * h)
        acc = jnp.maximum(acc, h)
    out_ref[0] = acc


def kernel(x, W_op1, W_op11, W_conv1, gamma1, beta1):
    q = pl.pallas_call(
        _pd_body,
        grid=(_B, _N // _R),
        in_specs=[pl.BlockSpec((1, _C, _N), lambda b, t: (b, 0, 0))],
        out_specs=pl.BlockSpec((1, _R, _NW2), lambda b, t: (b, t, 0)),
        out_shape=jax.ShapeDtypeStruct((_B, _N, _NW2), jnp.float32),
    )(x)
    q2 = q.reshape(_B * _N * _NW2)

    mesh = plsc.VectorSubcoreMesh(core_axis_name="c", subcore_axis_name="s")
    sc = pl.kernel(
        _sc_body,
        out_type=[jax.ShapeDtypeStruct((_B * _N * _OW,), jnp.float32)],
        mesh=mesh,
        compiler_params=pltpu.CompilerParams(needs_layout_passes=False),
        scratch_types=[
            pltpu.VMEM((2 * _BATCH * _NW2,), jnp.float32),
            pltpu.VMEM((128,), jnp.float32),
            pltpu.VMEM((128,), jnp.int32),
            pltpu.VMEM((2 * _BATCH * _OW,), jnp.float32),
            pltpu.VMEM((128,), jnp.int32),
            pltpu.VMEM((_C, _N), jnp.float32),
            pltpu.SemaphoreType.DMA,
            pltpu.SemaphoreType.DMA,
        ],
    )
    (scout,) = sc(q2, x)
    scout = scout.reshape(_B, _N, _OW)

    out = pl.pallas_call(
        _ec_body,
        grid=(_B, _N // _R),
        in_specs=[
            pl.BlockSpec((1, _C, _N), lambda b, t: (b, 0, 0)),
            pl.BlockSpec((1, _R, _OW), lambda b, t: (b, t, 0)),
            pl.BlockSpec((64, _DK), lambda b, t: (0, 0)),
            pl.BlockSpec((1, 64), lambda b, t: (0, 0)),
            pl.BlockSpec((64, 2 * _C), lambda b, t: (0, 0)),
            pl.BlockSpec((1, 64), lambda b, t: (0, 0)),
            pl.BlockSpec((1, 64), lambda b, t: (0, 0)),
        ],
        out_specs=pl.BlockSpec((1, _R, 64), lambda b, t: (b, t, 0)),
        out_shape=jax.ShapeDtypeStruct((_B, _N, 64), jnp.float32),
    )(x, scout, W_op1, W_op11, W_conv1,
      gamma1.reshape(1, 64), beta1.reshape(1, 64))
    return jnp.transpose(out, (0, 2, 1))


# speedup vs baseline: 1.0944x; 1.0944x over previous
import jax
import jax.numpy as jnp
from jax import lax
from jax.experimental import pallas as pl
from jax.experimental.pallas import tpu as pltpu
from jax.experimental.pallas import tpu_sc as plsc

_B, _C, _N = 8, 3, 2048
_DK, _K = 100, 20
_R = 128
_NW = 32
_ROWS_W = _B * _N // _NW
_BATCH = 4
_NBAT = _ROWS_W // _BATCH
_OW = 128 + 480
_NW2 = _N + 128
_POS = float("inf")
_CAND = 768


def _split3(a):
    hi = a.astype(jnp.bfloat16)
    r1 = a - hi.astype(jnp.float32)
    mid = r1.astype(jnp.bfloat16)
    lo = (r1 - mid.astype(jnp.float32)).astype(jnp.bfloat16)
    return hi, mid, lo


def _pd_body(x_ref, pd_ref):
    t = pl.program_id(1)
    xb = x_ref[0]
    xx_cols = xb[0:1] * xb[0:1] + xb[1:2] * xb[1:2] + xb[2:3] * xb[2:3]

    lane_n = lax.broadcasted_iota(jnp.int32, (_R, _N), 1)
    row_r = lax.broadcasted_iota(jnp.int32, (_R, 1), 0)

    xhi, xmid, xlo = _split3(xb)
    qhi, qmid, qlo = _split3(xx_cols)
    x12 = jnp.concatenate([xhi, xmid, xlo, qhi, qmid, qlo], axis=0)

    oh_self = (lane_n == t * _R + row_r).astype(jnp.bfloat16)
    g12 = lax.dot_general(oh_self, x12, (((1,), (1,)), ((), ())),
                          preferred_element_type=jnp.float32)
    xtT = g12[:, 0:3] + g12[:, 3:6] + g12[:, 6:9]
    xx_rows = g12[:, 9:10] + g12[:, 10:11] + g12[:, 11:12]

    inner = -2.0 * lax.dot_general(xtT.astype(jnp.bfloat16), xb.astype(jnp.bfloat16),
                                   (((1,), (0,)), ((), ())),
                                   preferred_element_type=jnp.float32)
    q = (xx_cols + inner) + xx_rows

    q3 = q.reshape(_R, 16, 128)
    for _ in range(8):
        m = jnp.min(q3, axis=2, keepdims=True)
        q3 = jnp.where(q3 == m, _POS, q3)
    tau = jnp.max(m.reshape(_R, 16), axis=1, keepdims=True)
    pd_ref[0, :, 0:_N] = q
    pd_ref[0, :, _N:_NW2] = jnp.broadcast_to(tau, (_R, 128))


def _rev(v):
    return lax.rev(v, (0,))


def _sort16(v, i):
    return lax.sort((v, i), dimension=0, num_keys=1, is_stable=False)


def _cmpsel(av, ai, bv, bi):
    bw = (bv < av) | ((bv == av) & (bi < ai))
    hv = jnp.where(bw, bv, av)
    hi_ = jnp.where(bw, bi, ai)
    lv = jnp.where(bw, av, bv)
    li = jnp.where(bw, ai, bi)
    return hv, hi_, lv, li


def _bmerge(vs, js):
    m = len(vs)
    if m == 1:
        v, j = _sort16(vs[0], js[0])
        return [v], [j]
    h = m // 2
    hv, hj, lv, lj = [], [], [], []
    for k in range(h):
        a, b, c, d = _cmpsel(vs[k], js[k], vs[k + h], js[k + h])
        hv.append(a); hj.append(b); lv.append(c); lj.append(d)
    rv1, rj1 = _bmerge(hv, hj)
    rv2, rj2 = _bmerge(lv, lj)
    return rv1 + rv2, rj1 + rj2


def _merge(av, aj, bv, bj):
    m = len(av)
    brv = [_rev(bv[m - 1 - k]) for k in range(m)]
    brj = [_rev(bj[m - 1 - k]) for k in range(m)]
    hv, hj, lv, lj = [], [], [], []
    for k in range(m):
        a, b, c, d = _cmpsel(av[k], aj[k], brv[k], brj[k])
        hv.append(a); hj.append(b); lv.append(c); lj.append(d)
    rv1, rj1 = _bmerge(hv, hj)
    rv2, rj2 = _bmerge(lv, lj)
    return rv1 + rv2, rj1 + rj2


def _sortN(vs, js):
    n = len(vs)
    if n == 1:
        v, j = _sort16(vs[0], js[0])
        return [v], [j]
    h = n // 2
    av, aj = _sortN(vs[:h], js[:h])
    bv, bj = _sortN(vs[h:], js[h:])
    return _merge(av, aj, bv, bj)


def _prune_merge(sv, sj, cv, cj):
    hv, hj = [], []
    for k in range(8):
        a, b, _, _ = _cmpsel(sv[k], sj[k], _rev(cv[7 - k]), _rev(cj[7 - k]))
        hv.append(a); hj.append(b)
    return _bmerge(hv, hj)


def _sc_body(pd_hbm, x_hbm, out_hbm,
             rowbuf, candv, candi, st, si_st, xbuf, sem_in, sem_out):
    wid = lax.axis_index("s") * 2 + lax.axis_index("c")
    row0 = wid * _ROWS_W
    b = row0 // _N
    pltpu.sync_copy(x_hbm.at[b], xbuf)

    iota16 = lax.iota(jnp.int32, 16)

    def cp_in(i, par):
        return pltpu.make_async_copy(
            pd_hbm.at[pl.ds((row0 + _BATCH * i) * _NW2, _BATCH * _NW2)],
            rowbuf.at[pl.ds(par * _BATCH * _NW2, _BATCH * _NW2)], sem_in)

    def cp_out(i, par):
        return pltpu.make_async_copy(
            st.at[pl.ds(par * _BATCH * _OW, _BATCH * _OW)],
            out_hbm.at[pl.ds((row0 + _BATCH * i) * _OW, _BATCH * _OW)], sem_out)

    def flat(sv, sj):
        return tuple(sv) + tuple(sj)

    def unflat(t):
        return list(t[:8]), list(t[8:16])

    def process_row(par, q):
        rbase = par * _BATCH * _NW2 + q * _NW2
        tauvec = rowbuf[pl.ds(rbase + _N, 16)]
        tau = tauvec[0]

        def blk(j, cnt):
            v = rowbuf[pl.ds(rbase + 16 * j, 16)]
            iv = iota16 + 16 * j
            mask = v <= tau
            pos = plsc.cumsum(mask.astype(jnp.int32))
            tgt = jnp.minimum(cnt + pos - 1, _CAND - 1)
            plsc.store_scatter(candv, [tgt], v, mask=mask)
            plsc.store_scatter(candi, [tgt], iv, mask=mask)
            return cnt + pos[15]

        cnt = lax.fori_loop(0, _N // 16, blk, jnp.int32(0), unroll=8)

        for tpad in range(8):
            tgt = jnp.minimum(cnt + iota16 + 16 * tpad, _CAND - 1)
            plsc.store_scatter(candv, [tgt], jnp.full((16,), _POS, jnp.float32))
            plsc.store_scatter(candi, [tgt], jnp.full((16,), _N - 1, jnp.int32))

        sv = [jnp.full((16,), _POS, jnp.float32) for _ in range(8)]
        sj = [jnp.full((16,), _N - 1, jnp.int32) for _ in range(8)]

        def mk_merge(c):
            def m(args):
                t8, cnt_c = args
                svx, sjx = unflat(t8)
                cv = [candv[pl.ds(128 * c + 16 * k, 16)] for k in range(8)]
                ci = [candi[pl.ds(128 * c + 16 * k, 16)] for k in range(8)]
                cv, ci = _sortN(cv, ci)
                svx, sjx = _prune_merge(svx, sjx, cv, ci)
                return flat(svx, sjx), cnt_c
            return m

        carry = (flat(sv, sj), cnt)
        for c in range(_CAND // 128):
            carry = lax.cond(carry[1] > 128 * c, mk_merge(c), lambda a: a, carry)
        sv, sj = unflat(carry[0])

        obase = par * _BATCH * _OW + q * _OW
        for k in range(8):
            st[pl.ds(obase + 16 * k, 16)] = sv[k]
            si_st[pl.ds(16 * k, 16)] = sj[k]

        for v in range(1, 6):
            for grp in range(2):
                posv = jnp.minimum((iota16 + 16 * grp) * v, 127)
                nidx = plsc.load_gather(si_st, [posv])
                for c in range(_C):
                    cvec = jnp.full((16,), c, jnp.int32)
                    xs = plsc.load_gather(xbuf, [cvec, nidx])
                    st[pl.ds(obase + 128 + (v - 1) * 96 + c * 32 + grp * 16, 16)] = xs

    cp_in(0, 0).start()

    def batch(i, _):
        par = lax.rem(i, 2)

        @pl.when(i + 1 < _NBAT)
        def _():
            cp_in(i + 1, 1 - par).start()

        cp_in(i, par).wait()

        @pl.when(i >= 2)
        def _():
            cp_out(i - 2, par).wait()

        def inner(q, __):
            process_row(par, q)
            return 0

        lax.fori_loop(0, _BATCH, inner, 0)
        cp_out(i, par).start()
        return 0

    lax.fori_loop(0, _NBAT, batch, 0)
    cp_out(_NBAT - 2, lax.rem(jnp.int32(_NBAT - 2), 2)).wait()
    cp_out(_NBAT - 1, lax.rem(jnp.int32(_NBAT - 1), 2)).wait()


def _ec_body(x_ref, sc_ref, w1_ref, w11_ref, wc_ref, g_ref, b_ref, out_ref):
    t = pl.program_id(1)
    xb = x_ref[0]
    lane_n = lax.broadcasted_iota(jnp.int32, (_R, _N), 1)
    row_r = lax.broadcasted_iota(jnp.int32, (_R, 1), 0)
    xhi, xmid, xlo = _split3(xb)
    x9 = jnp.concatenate([xhi, xmid, xlo], axis=0)
    oh_self = (lane_n == t * _R + row_r).astype(jnp.bfloat16)
    g9 = lax.dot_general(oh_self, x9, (((1,), (1,)), ((), ())),
                         preferred_element_type=jnp.float32)
    xtT = g9[:, 0:3] + g9[:, 3:6] + g9[:, 6:9]

    metric = sc_ref[0][:, 0:128]
    w1pad = jnp.concatenate(
        [w1_ref[...], jnp.zeros((64, 128 - _DK), jnp.float32)], axis=1)
    m1 = lax.dot_general(metric.astype(jnp.bfloat16), w1pad.astype(jnp.bfloat16),
                         (((1,), (1,)), ((), ())),
                         preferred_element_type=jnp.float32)
    w11pad = jnp.concatenate(
        [w11_ref[...], jnp.zeros((7, 64), jnp.float32)], axis=0)
    m2 = lax.dot_general(m1.astype(jnp.bfloat16), w11pad.astype(jnp.bfloat16),
                         (((1,), (1,)), ((), ())),
                         preferred_element_type=jnp.float32)[:, 0:1]
    ms = 5.0 * jax.nn.sigmoid(-m2) + 0.5
    value = (jnp.where((ms >= 0.5) & (ms < 1.5), 1.0, 0.0)
             + jnp.where((ms >= 1.5) & (ms < 2.5), 2.0, 0.0)
             + jnp.where((ms >= 2.5) & (ms < 3.5), 3.0, 0.0)
             + jnp.where((ms >= 3.5) & (ms < 4.5), 4.0, 0.0)
             + jnp.where((ms >= 4.5) & (ms <= 5.5), 5.0, 0.0))

    nb = sc_ref[0][:, 128:_OW]
    sel = jnp.zeros((_R, 96), jnp.float32)
    for v in range(1, 6):
        sel = jnp.where(value == jnp.float32(v), nb[:, 96 * (v - 1):96 * v], sel)

    wc_bf = wc_ref[...].astype(jnp.bfloat16)
    gamma = g_ref[...]
    beta = b_ref[...]
    acc = jnp.full((_R, 64), float("-inf"), jnp.float32)
    for k in range(_K):
        n0 = sel[:, k:k + 1]
        n1 = sel[:, 32 + k:33 + k]
        n2 = sel[:, 64 + k:65 + k]
        feat = jnp.concatenate(
            [n0 - xtT[:, 0:1], n1 - xtT[:, 1:2], n2 - xtT[:, 2:3], xtT], axis=1)
        h = lax.dot_general(feat.astype(jnp.bfloat16), wc_bf,
                            (((1,), (1,)), ((), ())),
                            preferred_element_type=jnp.float32)
        h = h * gamma + beta
        h = jnp.where(h >= 0, h, 0.2 * h)
        acc = jnp.maximum(acc, h)
    out_ref[0] = acc


def kernel(x, W_op1, W_op11, W_conv1, gamma1, beta1):
    q = pl.pallas_call(
        _pd_body,
        grid=(_B, _N // _R),
        in_specs=[pl.BlockSpec((1, _C, _N), lambda b, t: (b, 0, 0))],
        out_specs=pl.BlockSpec((1, _R, _NW2), lambda b, t: (b, t, 0)),
        out_shape=jax.ShapeDtypeStruct((_B, _N, _NW2), jnp.float32),
    )(x)
    q2 = q.reshape(_B * _N * _NW2)

    mesh = plsc.VectorSubcoreMesh(core_axis_name="c", subcore_axis_name="s")
    sc = pl.kernel(
        _sc_body,
        out_type=[jax.ShapeDtypeStruct((_B * _N * _OW,), jnp.float32)],
        mesh=mesh,
        compiler_params=pltpu.CompilerParams(needs_layout_passes=False),
        scratch_types=[
            pltpu.VMEM((2 * _BATCH * _NW2,), jnp.float32),
            pltpu.VMEM((_CAND,), jnp.float32),
            pltpu.VMEM((_CAND,), jnp.int32),
            pltpu.VMEM((2 * _BATCH * _OW,), jnp.float32),
            pltpu.VMEM((128,), jnp.int32),
            pltpu.VMEM((_C, _N), jnp.float32),
            pltpu.SemaphoreType.DMA,
            pltpu.SemaphoreType.DMA,
        ],
    )
    (scout,) = sc(q2, x)
    scout = scout.reshape(_B, _N, _OW)

    out = pl.pallas_call(
        _ec_body,
        grid=(_B, _N // _R),
        in_specs=[
            pl.BlockSpec((1, _C, _N), lambda b, t: (b, 0, 0)),
            pl.BlockSpec((1, _R, _OW), lambda b, t: (b, t, 0)),
            pl.BlockSpec((64, _DK), lambda b, t: (0, 0)),
            pl.BlockSpec((1, 64), lambda b, t: (0, 0)),
            pl.BlockSpec((64, 2 * _C), lambda b, t: (0, 0)),
            pl.BlockSpec((1, 64), lambda b, t: (0, 0)),
            pl.BlockSpec((1, 64), lambda b, t: (0, 0)),
        ],
        out_specs=pl.BlockSpec((1, _R, 64), lambda b, t: (b, t, 0)),
        out_shape=jax.ShapeDtypeStruct((_B, _N, 64), jnp.float32),
    )(x, scout, W_op1, W_op11, W_conv1,
      gamma1.reshape(1, 64), beta1.reshape(1, 64))
    return jnp.transpose(out, (0, 2, 1))


# speedup vs baseline: 1.3304x; 1.2156x over previous
import jax
import jax.numpy as jnp
from jax import lax
from jax.experimental import pallas as pl
from jax.experimental.pallas import tpu as pltpu
from jax.experimental.pallas import tpu_sc as plsc

_B, _C, _N = 8, 3, 2048
_DK, _K = 100, 20
_R = 128
_NW = 32
_ROWS_W = _B * _N // _NW
_BATCH = 4
_NBAT = _ROWS_W // _BATCH
_OW = 128 + 480
_NW2 = _N + 128
_POS = float("inf")
_CAND = 768


def _split3(a):
    hi = a.astype(jnp.bfloat16)
    r1 = a - hi.astype(jnp.float32)
    mid = r1.astype(jnp.bfloat16)
    lo = (r1 - mid.astype(jnp.float32)).astype(jnp.bfloat16)
    return hi, mid, lo


def _pd_body(x_ref, pd_ref):
    t = pl.program_id(1)
    xb = x_ref[0]
    xx_cols = xb[0:1] * xb[0:1] + xb[1:2] * xb[1:2] + xb[2:3] * xb[2:3]

    lane_n = lax.broadcasted_iota(jnp.int32, (_R, _N), 1)
    row_r = lax.broadcasted_iota(jnp.int32, (_R, 1), 0)

    xhi, xmid, xlo = _split3(xb)
    qhi, qmid, qlo = _split3(xx_cols)
    x12 = jnp.concatenate([xhi, xmid, xlo, qhi, qmid, qlo], axis=0)

    oh_self = (lane_n == t * _R + row_r).astype(jnp.bfloat16)
    g12 = lax.dot_general(oh_self, x12, (((1,), (1,)), ((), ())),
                          preferred_element_type=jnp.float32)
    xtT = g12[:, 0:3] + g12[:, 3:6] + g12[:, 6:9]
    xx_rows = g12[:, 9:10] + g12[:, 10:11] + g12[:, 11:12]

    inner = -2.0 * lax.dot_general(xtT.astype(jnp.bfloat16), xb.astype(jnp.bfloat16),
                                   (((1,), (0,)), ((), ())),
                                   preferred_element_type=jnp.float32)
    q = (xx_cols + inner) + xx_rows

    q3 = q.reshape(_R, 16, 128)
    for _ in range(8):
        m = jnp.min(q3, axis=2, keepdims=True)
        q3 = jnp.where(q3 == m, _POS, q3)
    tau = jnp.max(m.reshape(_R, 16), axis=1, keepdims=True)
    pd_ref[0, :, 0:_N] = q
    pd_ref[0, :, _N:_NW2] = jnp.broadcast_to(tau, (_R, 128))


def _rev(v):
    return lax.rev(v, (0,))


def _sort16(v, i):
    return lax.sort((v, i), dimension=0, num_keys=1, is_stable=False)


def _cmpsel(av, ai, bv, bi):
    bw = (bv < av) | ((bv == av) & (bi < ai))
    hv = jnp.where(bw, bv, av)
    hi_ = jnp.where(bw, bi, ai)
    lv = jnp.where(bw, av, bv)
    li = jnp.where(bw, ai, bi)
    return hv, hi_, lv, li


def _bmerge(vs, js):
    m = len(vs)
    if m == 1:
        v, j = _sort16(vs[0], js[0])
        return [v], [j]
    h = m // 2
    hv, hj, lv, lj = [], [], [], []
    for k in range(h):
        a, b, c, d = _cmpsel(vs[k], js[k], vs[k + h], js[k + h])
        hv.append(a); hj.append(b); lv.append(c); lj.append(d)
    rv1, rj1 = _bmerge(hv, hj)
    rv2, rj2 = _bmerge(lv, lj)
    return rv1 + rv2, rj1 + rj2


def _merge(av, aj, bv, bj):
    m = len(av)
    brv = [_rev(bv[m - 1 - k]) for k in range(m)]
    brj = [_rev(bj[m - 1 - k]) for k in range(m)]
    hv, hj, lv, lj = [], [], [], []
    for k in range(m):
        a, b, c, d = _cmpsel(av[k], aj[k], brv[k], brj[k])
        hv.append(a); hj.append(b); lv.append(c); lj.append(d)
    rv1, rj1 = _bmerge(hv, hj)
    rv2, rj2 = _bmerge(lv, lj)
    return rv1 + rv2, rj1 + rj2


def _sortN(vs, js):
    n = len(vs)
    if n == 1:
        v, j = _sort16(vs[0], js[0])
        return [v], [j]
    h = n // 2
    av, aj = _sortN(vs[:h], js[:h])
    bv, bj = _sortN(vs[h:], js[h:])
    return _merge(av, aj, bv, bj)


def _prune_merge(sv, sj, cv, cj):
    hv, hj = [], []
    for k in range(8):
        a, b, _, _ = _cmpsel(sv[k], sj[k], _rev(cv[7 - k]), _rev(cj[7 - k]))
        hv.append(a); hj.append(b)
    return _bmerge(hv, hj)


def _make_sc_body(rows_w):
  nbat = rows_w // _BATCH

  def _sc_body(pd_hbm, x_hbm, out_hbm,
               rowbuf, candv, candi, st, si_st, xbuf, sem_in, sem_out):
    wid = lax.axis_index("s") * 2 + lax.axis_index("c")
    row0 = wid * rows_w
    b = row0 // _N
    pltpu.sync_copy(x_hbm.at[b], xbuf)

    iota16 = lax.iota(jnp.int32, 16)

    def cp_in(i, par):
        return pltpu.make_async_copy(
            pd_hbm.at[pl.ds((row0 + _BATCH * i) * _NW2, _BATCH * _NW2)],
            rowbuf.at[pl.ds(par * _BATCH * _NW2, _BATCH * _NW2)], sem_in)

    def cp_out(i, par):
        return pltpu.make_async_copy(
            st.at[pl.ds(par * _BATCH * _OW, _BATCH * _OW)],
            out_hbm.at[pl.ds((row0 + _BATCH * i) * _OW, _BATCH * _OW)], sem_out)

    def flat(sv, sj):
        return tuple(sv) + tuple(sj)

    def unflat(t):
        return list(t[:8]), list(t[8:16])

    def process_row(par, q):
        rbase = par * _BATCH * _NW2 + q * _NW2
        tauvec = rowbuf[pl.ds(rbase + _N, 16)]
        tau = tauvec[0]

        def blk(j, cnt):
            v = rowbuf[pl.ds(rbase + 16 * j, 16)]
            iv = iota16 + 16 * j
            mask = v <= tau
            pos = plsc.cumsum(mask.astype(jnp.int32))
            tgt = jnp.minimum(cnt + pos - 1, _CAND - 1)
            plsc.store_scatter(candv, [tgt], v, mask=mask)
            plsc.store_scatter(candi, [tgt], iv, mask=mask)
            return cnt + pos[15]

        cnt = lax.fori_loop(0, _N // 16, blk, jnp.int32(0), unroll=8)

        for tpad in range(8):
            tgt = jnp.minimum(cnt + iota16 + 16 * tpad, _CAND - 1)
            plsc.store_scatter(candv, [tgt], jnp.full((16,), _POS, jnp.float32))
            plsc.store_scatter(candi, [tgt], jnp.full((16,), _N - 1, jnp.int32))

        sv = [jnp.full((16,), _POS, jnp.float32) for _ in range(8)]
        sj = [jnp.full((16,), _N - 1, jnp.int32) for _ in range(8)]

        def mk_merge(c):
            def m(args):
                t8, cnt_c = args
                svx, sjx = unflat(t8)
                cv = [candv[pl.ds(128 * c + 16 * k, 16)] for k in range(8)]
                ci = [candi[pl.ds(128 * c + 16 * k, 16)] for k in range(8)]
                cv, ci = _sortN(cv, ci)
                svx, sjx = _prune_merge(svx, sjx, cv, ci)
                return flat(svx, sjx), cnt_c
            return m

        carry = (flat(sv, sj), cnt)
        for c in range(_CAND // 128):
            carry = lax.cond(carry[1] > 128 * c, mk_merge(c), lambda a: a, carry)
        sv, sj = unflat(carry[0])

        obase = par * _BATCH * _OW + q * _OW
        for k in range(8):
            st[pl.ds(obase + 16 * k, 16)] = sv[k]
            si_st[pl.ds(16 * k, 16)] = sj[k]

        for v in range(1, 6):
            for grp in range(2):
                posv = jnp.minimum((iota16 + 16 * grp) * v, 127)
                nidx = plsc.load_gather(si_st, [posv])
                for c in range(_C):
                    cvec = jnp.full((16,), c, jnp.int32)
                    xs = plsc.load_gather(xbuf, [cvec, nidx])
                    st[pl.ds(obase + 128 + (v - 1) * 96 + c * 32 + grp * 16, 16)] = xs

    cp_in(0, 0).start()

    def batch(i, _):
        par = lax.rem(i, 2)

        @pl.when(i + 1 < nbat)
        def _():
            cp_in(i + 1, 1 - par).start()

        cp_in(i, par).wait()

        @pl.when(i >= 2)
        def _():
            cp_out(i - 2, par).wait()

        def inner(q, __):
            process_row(par, q)
            return 0

        lax.fori_loop(0, _BATCH, inner, 0)
        cp_out(i, par).start()
        return 0

    lax.fori_loop(0, nbat, batch, 0)
    cp_out(nbat - 2, lax.rem(jnp.int32(nbat - 2), 2)).wait()
    cp_out(nbat - 1, lax.rem(jnp.int32(nbat - 1), 2)).wait()

  return _sc_body


def _ec_body(x_ref, sc_ref, w1_ref, w11_ref, wc_ref, g_ref, b_ref, out_ref):
    t = pl.program_id(1)
    xb = x_ref[0]
    lane_n = lax.broadcasted_iota(jnp.int32, (_R, _N), 1)
    row_r = lax.broadcasted_iota(jnp.int32, (_R, 1), 0)
    xhi, xmid, xlo = _split3(xb)
    x9 = jnp.concatenate([xhi, xmid, xlo], axis=0)
    oh_self = (lane_n == t * _R + row_r).astype(jnp.bfloat16)
    g9 = lax.dot_general(oh_self, x9, (((1,), (1,)), ((), ())),
                         preferred_element_type=jnp.float32)
    xtT = g9[:, 0:3] + g9[:, 3:6] + g9[:, 6:9]

    metric = sc_ref[0][:, 0:128]
    w1pad = jnp.concatenate(
        [w1_ref[...], jnp.zeros((64, 128 - _DK), jnp.float32)], axis=1)
    m1 = lax.dot_general(metric.astype(jnp.bfloat16), w1pad.astype(jnp.bfloat16),
                         (((1,), (1,)), ((), ())),
                         preferred_element_type=jnp.float32)
    w11pad = jnp.concatenate(
        [w11_ref[...], jnp.zeros((7, 64), jnp.float32)], axis=0)
    m2 = lax.dot_general(m1.astype(jnp.bfloat16), w11pad.astype(jnp.bfloat16),
                         (((1,), (1,)), ((), ())),
                         preferred_element_type=jnp.float32)[:, 0:1]
    ms = 5.0 * jax.nn.sigmoid(-m2) + 0.5
    value = (jnp.where((ms >= 0.5) & (ms < 1.5), 1.0, 0.0)
             + jnp.where((ms >= 1.5) & (ms < 2.5), 2.0, 0.0)
             + jnp.where((ms >= 2.5) & (ms < 3.5), 3.0, 0.0)
             + jnp.where((ms >= 3.5) & (ms < 4.5), 4.0, 0.0)
             + jnp.where((ms >= 4.5) & (ms <= 5.5), 5.0, 0.0))

    nb = sc_ref[0][:, 128:_OW]
    sel = jnp.zeros((_R, 96), jnp.float32)
    for v in range(1, 6):
        sel = jnp.where(value == jnp.float32(v), nb[:, 96 * (v - 1):96 * v], sel)

    wc_bf = wc_ref[...].astype(jnp.bfloat16)
    gamma = g_ref[...]
    beta = b_ref[...]
    acc = jnp.full((_R, 64), float("-inf"), jnp.float32)
    for k in range(_K):
        n0 = sel[:, k:k + 1]
        n1 = sel[:, 32 + k:33 + k]
        n2 = sel[:, 64 + k:65 + k]
        feat = jnp.concatenate(
            [n0 - xtT[:, 0:1], n1 - xtT[:, 1:2], n2 - xtT[:, 2:3], xtT], axis=1)
        h = lax.dot_general(feat.astype(jnp.bfloat16), wc_bf,
                            (((1,), (1,)), ((), ())),
                            preferred_element_type=jnp.float32)
        h = h * gamma + beta
        h = jnp.where(h >= 0, h, 0.2 * h)
        acc = jnp.maximum(acc, h)
    out_ref[0] = acc


def kernel(x, W_op1, W_op11, W_conv1, gamma1, beta1):
    bh = _B // 2
    rows_w = bh * _N // _NW

    mesh = plsc.VectorSubcoreMesh(core_axis_name="c", subcore_axis_name="s")
    sc = pl.kernel(
        _make_sc_body(rows_w),
        out_type=[jax.ShapeDtypeStruct((bh * _N * _OW,), jnp.float32)],
        mesh=mesh,
        compiler_params=pltpu.CompilerParams(needs_layout_passes=False),
        scratch_types=[
            pltpu.VMEM((2 * _BATCH * _NW2,), jnp.float32),
            pltpu.VMEM((_CAND,), jnp.float32),
            pltpu.VMEM((_CAND,), jnp.int32),
            pltpu.VMEM((2 * _BATCH * _OW,), jnp.float32),
            pltpu.VMEM((128,), jnp.int32),
            pltpu.VMEM((_C, _N), jnp.float32),
            pltpu.SemaphoreType.DMA,
            pltpu.SemaphoreType.DMA,
        ],
    )

    scouts = []
    xhs = []
    for h in range(2):
        xh = lax.slice_in_dim(x, bh * h, bh * (h + 1), axis=0)
        xhs.append(xh)
        q = pl.pallas_call(
            _pd_body,
            grid=(bh, _N // _R),
            in_specs=[pl.BlockSpec((1, _C, _N), lambda b, t: (b, 0, 0))],
            out_specs=pl.BlockSpec((1, _R, _NW2), lambda b, t: (b, t, 0)),
            out_shape=jax.ShapeDtypeStruct((bh, _N, _NW2), jnp.float32),
        )(xh)
        (scout,) = sc(q.reshape(bh * _N * _NW2), xh)
        scouts.append(scout.reshape(bh, _N, _OW))

    outs = []
    for h in range(2):
        out = pl.pallas_call(
            _ec_body,
            grid=(bh, _N // _R),
            in_specs=[
                pl.BlockSpec((1, _C, _N), lambda b, t: (b, 0, 0)),
                pl.BlockSpec((1, _R, _OW), lambda b, t: (b, t, 0)),
                pl.BlockSpec((64, _DK), lambda b, t: (0, 0)),
                pl.BlockSpec((1, 64), lambda b, t: (0, 0)),
                pl.BlockSpec((64, 2 * _C), lambda b, t: (0, 0)),
                pl.BlockSpec((1, 64), lambda b, t: (0, 0)),
                pl.BlockSpec((1, 64), lambda b, t: (0, 0)),
            ],
            out_specs=pl.BlockSpec((1, _R, 64), lambda b, t: (b, t, 0)),
            out_shape=jax.ShapeDtypeStruct((bh, _N, 64), jnp.float32),
        )(xhs[h], scouts[h], W_op1, W_op11, W_conv1,
          gamma1.reshape(1, 64), beta1.reshape(1, 64))
        outs.append(out)
    return jnp.transpose(jnp.concatenate(outs, axis=0), (0, 2, 1))


# speedup vs baseline: 1.4764x; 1.1097x over previous
import jax
import jax.numpy as jnp
from jax import lax
from jax.experimental import pallas as pl
from jax.experimental.pallas import tpu as pltpu
from jax.experimental.pallas import tpu_sc as plsc

_B, _C, _N = 8, 3, 2048
_DK, _K = 100, 20
_R = 128
_NW = 32
_ROWS_W = _B * _N // _NW
_BATCH = 4
_NBAT = _ROWS_W // _BATCH
_OW = 128 + 480
_NW2 = _N + 128
_POS = float("inf")
_CAND = 768


def _split3(a):
    hi = a.astype(jnp.bfloat16)
    r1 = a - hi.astype(jnp.float32)
    mid = r1.astype(jnp.bfloat16)
    lo = (r1 - mid.astype(jnp.float32)).astype(jnp.bfloat16)
    return hi, mid, lo


def _pd_body(x_ref, pd_ref):
    t = pl.program_id(1)
    xb = x_ref[0]
    xx_cols = xb[0:1] * xb[0:1] + xb[1:2] * xb[1:2] + xb[2:3] * xb[2:3]

    lane_n = lax.broadcasted_iota(jnp.int32, (_R, _N), 1)
    row_r = lax.broadcasted_iota(jnp.int32, (_R, 1), 0)

    xhi, xmid, xlo = _split3(xb)
    qhi, qmid, qlo = _split3(xx_cols)
    x12 = jnp.concatenate([xhi, xmid, xlo, qhi, qmid, qlo], axis=0)

    oh_self = (lane_n == t * _R + row_r).astype(jnp.bfloat16)
    g12 = lax.dot_general(oh_self, x12, (((1,), (1,)), ((), ())),
                          preferred_element_type=jnp.float32)
    xtT = g12[:, 0:3] + g12[:, 3:6] + g12[:, 6:9]
    xx_rows = g12[:, 9:10] + g12[:, 10:11] + g12[:, 11:12]

    inner = -2.0 * lax.dot_general(xtT.astype(jnp.bfloat16), xb.astype(jnp.bfloat16),
                                   (((1,), (0,)), ((), ())),
                                   preferred_element_type=jnp.float32)
    q = (xx_cols + inner) + xx_rows

    q3 = q.reshape(_R, 16, 128)
    for _ in range(8):
        m = jnp.min(q3, axis=2, keepdims=True)
        q3 = jnp.where(q3 == m, _POS, q3)
    tau = jnp.max(m.reshape(_R, 16), axis=1, keepdims=True)
    pd_ref[0, :, 0:_N] = q
    pd_ref[0, :, _N:_NW2] = jnp.broadcast_to(tau, (_R, 128))


def _rev(v):
    return lax.rev(v, (0,))


def _sort16(v, i):
    return lax.sort((v, i), dimension=0, num_keys=1, is_stable=False)


def _cmpsel(av, ai, bv, bi):
    bw = (bv < av) | ((bv == av) & (bi < ai))
    hv = jnp.where(bw, bv, av)
    hi_ = jnp.where(bw, bi, ai)
    lv = jnp.where(bw, av, bv)
    li = jnp.where(bw, ai, bi)
    return hv, hi_, lv, li


def _bmerge(vs, js):
    m = len(vs)
    if m == 1:
        v, j = _sort16(vs[0], js[0])
        return [v], [j]
    h = m // 2
    hv, hj, lv, lj = [], [], [], []
    for k in range(h):
        a, b, c, d = _cmpsel(vs[k], js[k], vs[k + h], js[k + h])
        hv.append(a); hj.append(b); lv.append(c); lj.append(d)
    rv1, rj1 = _bmerge(hv, hj)
    rv2, rj2 = _bmerge(lv, lj)
    return rv1 + rv2, rj1 + rj2


def _merge(av, aj, bv, bj):
    m = len(av)
    brv = [_rev(bv[m - 1 - k]) for k in range(m)]
    brj = [_rev(bj[m - 1 - k]) for k in range(m)]
    hv, hj, lv, lj = [], [], [], []
    for k in range(m):
        a, b, c, d = _cmpsel(av[k], aj[k], brv[k], brj[k])
        hv.append(a); hj.append(b); lv.append(c); lj.append(d)
    rv1, rj1 = _bmerge(hv, hj)
    rv2, rj2 = _bmerge(lv, lj)
    return rv1 + rv2, rj1 + rj2


def _sortN(vs, js):
    n = len(vs)
    if n == 1:
        v, j = _sort16(vs[0], js[0])
        return [v], [j]
    h = n // 2
    av, aj = _sortN(vs[:h], js[:h])
    bv, bj = _sortN(vs[h:], js[h:])
    return _merge(av, aj, bv, bj)


def _prune_merge(sv, sj, cv, cj):
    hv, hj = [], []
    for k in range(8):
        a, b, _, _ = _cmpsel(sv[k], sj[k], _rev(cv[7 - k]), _rev(cj[7 - k]))
        hv.append(a); hj.append(b)
    return _bmerge(hv, hj)


def _make_sc_body(rows_w):
  nbat = rows_w // _BATCH

  def _sc_body(pd_hbm, x_hbm, out_hbm,
               rowbuf, candv, candi, st, si_st, xbuf, sem_in, sem_out):
    wid = lax.axis_index("s") * 2 + lax.axis_index("c")
    row0 = wid * rows_w
    b = row0 // _N
    pltpu.sync_copy(x_hbm.at[b], xbuf)

    iota16 = lax.iota(jnp.int32, 16)

    def cp_in(i, par):
        return pltpu.make_async_copy(
            pd_hbm.at[pl.ds((row0 + _BATCH * i) * _NW2, _BATCH * _NW2)],
            rowbuf.at[pl.ds(par * _BATCH * _NW2, _BATCH * _NW2)], sem_in)

    def cp_out(i, par):
        return pltpu.make_async_copy(
            st.at[pl.ds(par * _BATCH * _OW, _BATCH * _OW)],
            out_hbm.at[pl.ds((row0 + _BATCH * i) * _OW, _BATCH * _OW)], sem_out)

    def flat(sv, sj):
        return tuple(sv) + tuple(sj)

    def unflat(t):
        return list(t[:8]), list(t[8:16])

    def process_row(par, q):
        rbase = par * _BATCH * _NW2 + q * _NW2
        tauvec = rowbuf[pl.ds(rbase + _N, 16)]
        tau = tauvec[0]

        def blk(j, cnt):
            v = rowbuf[pl.ds(rbase + 16 * j, 16)]
            iv = iota16 + 16 * j
            mask = v <= tau
            pos = plsc.cumsum(mask.astype(jnp.int32))
            tgt = jnp.minimum(cnt + pos - 1, _CAND - 1)
            plsc.store_scatter(candv, [tgt], v, mask=mask)
            plsc.store_scatter(candi, [tgt], iv, mask=mask)
            return cnt + pos[15]

        cnt = lax.fori_loop(0, _N // 16, blk, jnp.int32(0), unroll=8)

        for tpad in range(8):
            tgt = jnp.minimum(cnt + iota16 + 16 * tpad, _CAND - 1)
            plsc.store_scatter(candv, [tgt], jnp.full((16,), _POS, jnp.float32))
            plsc.store_scatter(candi, [tgt], jnp.full((16,), _N - 1, jnp.int32))

        sv = [jnp.full((16,), _POS, jnp.float32) for _ in range(8)]
        sj = [jnp.full((16,), _N - 1, jnp.int32) for _ in range(8)]

        def mk_merge(c):
            def m(args):
                t8, cnt_c = args
                svx, sjx = unflat(t8)
                cv = [candv[pl.ds(128 * c + 16 * k, 16)] for k in range(8)]
                ci = [candi[pl.ds(128 * c + 16 * k, 16)] for k in range(8)]
                cv, ci = _sortN(cv, ci)
                svx, sjx = _prune_merge(svx, sjx, cv, ci)
                return flat(svx, sjx), cnt_c
            return m

        carry = (flat(sv, sj), cnt)
        for c in range(_CAND // 128):
            carry = lax.cond(carry[1] > 128 * c, mk_merge(c), lambda a: a, carry)
        sv, sj = unflat(carry[0])

        obase = par * _BATCH * _OW + q * _OW
        for k in range(8):
            st[pl.ds(obase + 16 * k, 16)] = sv[k]
            si_st[pl.ds(16 * k, 16)] = sj[k]

        for v in range(1, 6):
            for grp in range(2):
                posv = jnp.minimum((iota16 + 16 * grp) * v, 127)
                nidx = plsc.load_gather(si_st, [posv])
                for c in range(_C):
                    cvec = jnp.full((16,), c, jnp.int32)
                    xs = plsc.load_gather(xbuf, [cvec, nidx])
                    st[pl.ds(obase + 128 + (v - 1) * 96 + c * 32 + grp * 16, 16)] = xs

    cp_in(0, 0).start()

    def batch(i, _):
        par = lax.rem(i, 2)

        @pl.when(i + 1 < nbat)
        def _():
            cp_in(i + 1, 1 - par).start()

        cp_in(i, par).wait()

        @pl.when(i >= 2)
        def _():
            cp_out(i - 2, par).wait()

        def inner(q, __):
            process_row(par, q)
            return 0

        lax.fori_loop(0, _BATCH, inner, 0)
        cp_out(i, par).start()
        return 0

    lax.fori_loop(0, nbat, batch, 0)
    cp_out(nbat - 2, lax.rem(jnp.int32(nbat - 2), 2)).wait()
    cp_out(nbat - 1, lax.rem(jnp.int32(nbat - 1), 2)).wait()

  return _sc_body


def _ec_body(x_ref, sc_ref, w1_ref, w11_ref, wc_ref, g_ref, b_ref, out_ref):
    t = pl.program_id(1)
    xb = x_ref[0]
    lane_n = lax.broadcasted_iota(jnp.int32, (_R, _N), 1)
    row_r = lax.broadcasted_iota(jnp.int32, (_R, 1), 0)
    xhi, xmid, xlo = _split3(xb)
    x9 = jnp.concatenate([xhi, xmid, xlo], axis=0)
    oh_self = (lane_n == t * _R + row_r).astype(jnp.bfloat16)
    g9 = lax.dot_general(oh_self, x9, (((1,), (1,)), ((), ())),
                         preferred_element_type=jnp.float32)
    xtT = g9[:, 0:3] + g9[:, 3:6] + g9[:, 6:9]

    metric = sc_ref[0][:, 0:128]
    w1pad = jnp.concatenate(
        [w1_ref[...], jnp.zeros((64, 128 - _DK), jnp.float32)], axis=1)
    m1 = lax.dot_general(metric.astype(jnp.bfloat16), w1pad.astype(jnp.bfloat16),
                         (((1,), (1,)), ((), ())),
                         preferred_element_type=jnp.float32)
    w11pad = jnp.concatenate(
        [w11_ref[...], jnp.zeros((7, 64), jnp.float32)], axis=0)
    m2 = lax.dot_general(m1.astype(jnp.bfloat16), w11pad.astype(jnp.bfloat16),
                         (((1,), (1,)), ((), ())),
                         preferred_element_type=jnp.float32)[:, 0:1]
    ms = 5.0 * jax.nn.sigmoid(-m2) + 0.5
    value = (jnp.where((ms >= 0.5) & (ms < 1.5), 1.0, 0.0)
             + jnp.where((ms >= 1.5) & (ms < 2.5), 2.0, 0.0)
             + jnp.where((ms >= 2.5) & (ms < 3.5), 3.0, 0.0)
             + jnp.where((ms >= 3.5) & (ms < 4.5), 4.0, 0.0)
             + jnp.where((ms >= 4.5) & (ms <= 5.5), 5.0, 0.0))

    nb = sc_ref[0][:, 128:_OW]
    sel = jnp.zeros((_R, 96), jnp.float32)
    for v in range(1, 6):
        sel = jnp.where(value == jnp.float32(v), nb[:, 96 * (v - 1):96 * v], sel)

    wc_bf = wc_ref[...].astype(jnp.bfloat16)
    gamma = g_ref[...]
    beta = b_ref[...]
    acc = jnp.full((_R, 64), float("-inf"), jnp.float32)
    for k in range(_K):
        n0 = sel[:, k:k + 1]
        n1 = sel[:, 32 + k:33 + k]
        n2 = sel[:, 64 + k:65 + k]
        feat = jnp.concatenate(
            [n0 - xtT[:, 0:1], n1 - xtT[:, 1:2], n2 - xtT[:, 2:3], xtT], axis=1)
        h = lax.dot_general(feat.astype(jnp.bfloat16), wc_bf,
                            (((1,), (1,)), ((), ())),
                            preferred_element_type=jnp.float32)
        h = h * gamma + beta
        h = jnp.where(h >= 0, h, 0.2 * h)
        acc = jnp.maximum(acc, h)
    out_ref[0] = acc


def kernel(x, W_op1, W_op11, W_conv1, gamma1, beta1):
    nsplit = 4
    bh = _B // nsplit
    rows_w = bh * _N // _NW

    mesh = plsc.VectorSubcoreMesh(core_axis_name="c", subcore_axis_name="s")
    sc = pl.kernel(
        _make_sc_body(rows_w),
        out_type=[jax.ShapeDtypeStruct((bh * _N * _OW,), jnp.float32)],
        mesh=mesh,
        compiler_params=pltpu.CompilerParams(needs_layout_passes=False),
        scratch_types=[
            pltpu.VMEM((2 * _BATCH * _NW2,), jnp.float32),
            pltpu.VMEM((_CAND,), jnp.float32),
            pltpu.VMEM((_CAND,), jnp.int32),
            pltpu.VMEM((2 * _BATCH * _OW,), jnp.float32),
            pltpu.VMEM((128,), jnp.int32),
            pltpu.VMEM((_C, _N), jnp.float32),
            pltpu.SemaphoreType.DMA,
            pltpu.SemaphoreType.DMA,
        ],
    )

    scouts = []
    xhs = []
    for h in range(nsplit):
        xh = lax.slice_in_dim(x, bh * h, bh * (h + 1), axis=0)
        xhs.append(xh)
        q = pl.pallas_call(
            _pd_body,
            grid=(bh, _N // _R),
            in_specs=[pl.BlockSpec((1, _C, _N), lambda b, t: (b, 0, 0))],
            out_specs=pl.BlockSpec((1, _R, _NW2), lambda b, t: (b, t, 0)),
            out_shape=jax.ShapeDtypeStruct((bh, _N, _NW2), jnp.float32),
        )(xh)
        (scout,) = sc(q.reshape(bh * _N * _NW2), xh)
        scouts.append(scout.reshape(bh, _N, _OW))

    outs = []
    for h in range(nsplit):
        out = pl.pallas_call(
            _ec_body,
            grid=(bh, _N // _R),
            in_specs=[
                pl.BlockSpec((1, _C, _N), lambda b, t: (b, 0, 0)),
                pl.BlockSpec((1, _R, _OW), lambda b, t: (b, t, 0)),
                pl.BlockSpec((64, _DK), lambda b, t: (0, 0)),
                pl.BlockSpec((1, 64), lambda b, t: (0, 0)),
                pl.BlockSpec((64, 2 * _C), lambda b, t: (0, 0)),
                pl.BlockSpec((1, 64), lambda b, t: (0, 0)),
                pl.BlockSpec((1, 64), lambda b, t: (0, 0)),
            ],
            out_specs=pl.BlockSpec((1, _R, 64), lambda b, t: (b, t, 0)),
            out_shape=jax.ShapeDtypeStruct((bh, _N, 64), jnp.float32),
        )(xhs[h], scouts[h], W_op1, W_op11, W_conv1,
          gamma1.reshape(1, 64), beta1.reshape(1, 64))
        outs.append(out)
    return jnp.transpose(jnp.concatenate(outs, axis=0), (0, 2, 1))
